# Initial kernel scaffold; baseline (speedup 1.0000x reference)
import jax
import jax.numpy as jnp
from jax.experimental import pallas as pl


def kernel(x, gso, weight, bias):
    raise NotImplementedError("write your pallas kernel here")


# speedup vs baseline: 1.0301x; 1.0301x over previous
import functools

import jax
import jax.numpy as jnp
from jax.experimental import pallas as pl
from jax.experimental.pallas import tpu as pltpu

_ROWS = 400


def _cheb_kernel(x_rows_ref, gso_ref, x_full_ref, w0_ref, w1_ref, bias_ref,
                 out_ref, xw1_ref):
    i = pl.program_id(0)

    @pl.when(i == 0)
    def _init():
        xw1_ref[...] = jnp.dot(x_full_ref[...], w1_ref[...],
                               preferred_element_type=jnp.float32)

    out_ref[...] = (
        jnp.dot(x_rows_ref[...], w0_ref[...],
                preferred_element_type=jnp.float32)
        + jnp.dot(gso_ref[...], xw1_ref[...],
                  preferred_element_type=jnp.float32)
        + bias_ref[...]
    )


@functools.partial(jax.jit, static_argnames=())
def kernel(x, gso, weight, bias):
    b, n, d_in = x.shape
    d_out = weight.shape[-1]
    x2 = x[0]
    gso2 = gso[0]
    w0 = weight[0]
    w1 = weight[1]
    bias2 = bias.reshape(1, d_out)

    grid = (n // _ROWS,)
    out = pl.pallas_call(
        _cheb_kernel,
        grid=grid,
        in_specs=[
            pl.BlockSpec((_ROWS, d_in), lambda i: (i, 0)),
            pl.BlockSpec((_ROWS, n), lambda i: (i, 0)),
            pl.BlockSpec((n, d_in), lambda i: (0, 0)),
            pl.BlockSpec((d_in, d_out), lambda i: (0, 0)),
            pl.BlockSpec((d_in, d_out), lambda i: (0, 0)),
            pl.BlockSpec((1, d_out), lambda i: (0, 0)),
        ],
        out_specs=pl.BlockSpec((_ROWS, d_out), lambda i: (i, 0)),
        out_shape=jax.ShapeDtypeStruct((n, d_out), jnp.float32),
        scratch_shapes=[pltpu.VMEM((n, d_out), jnp.float32)],
    )(x2, gso2, x2, w0, w1, bias2)
    return out.reshape(b, n, d_out)
